# baseline (speedup 1.0000x reference)
import jax
import jax.numpy as jnp
from jax import lax
from jax.experimental import pallas as pl
from jax.experimental.pallas import tpu as pltpu
from jax.experimental.pallas import tpu_sc as plsc

NC = 2
NS = 16
NW = NC * NS
CHUNK = 128
NSLOT = 2
PADR = 240


def _sc_segsum(n_pad, ch, feat, with_cnt):
  zrows = n_pad // NS
  ib = 16
  assert ch % ib == 0 and ib % NSLOT == 0

  mesh = plsc.VectorSubcoreMesh(
      core_axis_name="c", subcore_axis_name="s",
      num_cores=NC, num_subcores=NS)

  out_type = [jax.ShapeDtypeStruct((NC, n_pad, feat), jnp.float32)]
  scratch = [
      pltpu.VMEM((ib, CHUNK), jnp.int32),
      pltpu.VMEM((ib, CHUNK), jnp.int32),
      pltpu.VMEM((NSLOT, CHUNK, feat), jnp.float32),
      pltpu.VMEM_SHARED((n_pad, feat), jnp.float32),
      [pltpu.SemaphoreType.DMA] * NSLOT,
      [pltpu.SemaphoreType.DMA] * NSLOT,
  ]
  if with_cnt:
    out_type.append(jax.ShapeDtypeStruct((NC * n_pad,), jnp.float32))
    scratch += [
        pltpu.VMEM((CHUNK,), jnp.float32),
        pltpu.VMEM((1024,), jnp.float32),
        pltpu.VMEM_SHARED((n_pad,), jnp.float32),
    ]

  def body(h_hbm, src_hbm, dst_hbm, s_out, *rest):
    if with_cnt:
      (cnt_out, src_v, dst_v, buf, acc, sem_g, sem_s,
       ones_v, zeros_v, cnt_acc) = rest
    else:
      src_v, dst_v, buf, acc, sem_g, sem_s = rest
      cnt_out = ones_v = zeros_v = cnt_acc = None
    cid = lax.axis_index("c")
    sid = lax.axis_index("s")
    wid = sid * NC + cid

    def gather(l, u):
      return pltpu.make_async_copy(h_hbm.at[src_v.at[l]], buf.at[u],
                                   sem_g[u])

    def scatter(l, u):
      return pltpu.make_async_copy(buf.at[u], acc.at[dst_v.at[l]],
                                   sem_s[u])

    zero16 = jnp.zeros((16,), jnp.float32)

    def zrow(r, carry):
      def zcol(k, c2):
        buf[0, r, pl.ds(k * 16, 16)] = zero16
        return c2
      return lax.fori_loop(0, feat // 16, zcol, carry)
    lax.fori_loop(0, CHUNK, zrow, 0)

    base = sid * zrows
    nz = zrows // CHUNK

    def zdma(i, carry):
      pltpu.sync_copy(buf.at[0],
                      acc.at[pl.ds(base + i * CHUNK, CHUNK)])
      return carry
    lax.fori_loop(0, nz, zdma, 0)
    assert nz * CHUNK == zrows

    if with_cnt:
      one16 = jnp.ones((16,), jnp.float32)
      for k in range(CHUNK // 16):
        ones_v[pl.ds(k * 16, 16)] = one16

      def zh(i, carry):
        zeros_v[pl.ds(i * 16, 16)] = zero16
        return carry
      lax.fori_loop(0, 1024 // 16, zh, 0)

      @pl.when(sid == 0)
      def _():
        def zc(i, carry):
          pltpu.sync_copy(zeros_v, cnt_acc.at[pl.ds(i * 1024, 1024)])
          return carry
        lax.fori_loop(0, n_pad // 1024, zc, 0)
        assert n_pad % 1024 == 0

    plsc.subcore_barrier()

    def count_chunk(l):
      if with_cnt:
        pltpu.sync_copy(ones_v, cnt_acc.at[dst_v.at[l]], add=True)

    def group(g, carry):
      @pl.when(g > 0)
      def _():
        scatter(0, 0).wait()
        scatter(0, 1).wait()
      pltpu.sync_copy(src_hbm.at[wid, pl.ds(g * ib, ib)], src_v)
      pltpu.sync_copy(dst_hbm.at[wid, pl.ds(g * ib, ib)], dst_v)
      gather(0, 0).start()
      gather(1, 1).start()

      def pair(p, c2):
        for u in range(NSLOT):
          l = NSLOT * p + u
          gather(l, u).wait()
          scatter(l, u).start(add=True)
          count_chunk(l)
        for u in range(NSLOT):
          l = NSLOT * p + u
          lp = l + NSLOT

          @pl.when(lp < ib)
          def _():
            scatter(l, u).wait()
            gather(lp, u).start()
        return c2
      lax.fori_loop(0, ib // NSLOT, pair, 0)
      return carry
    lax.fori_loop(0, ch // ib, group, 0)

    for u in range(NSLOT):
      scatter(0, u).wait()
    plsc.subcore_barrier()
    pltpu.sync_copy(acc.at[pl.ds(base, zrows)],
                    s_out.at[cid, pl.ds(base, zrows)])
    if with_cnt:
      @pl.when(sid == 0)
      def _():
        pltpu.sync_copy(cnt_acc, cnt_out.at[pl.ds(cid * n_pad, n_pad)])

  return pl.kernel(body, out_type=tuple(out_type), mesh=mesh,
                   scratch_types=tuple(scratch))


def _tc_layer(n_pad, feat, hidden, relu, blk=1024):
  def body(s_ref, c_ref, h_ref, wl_ref, wr_ref, b_ref, o_ref):
    j = pl.program_id(0)
    s = s_ref[0] + s_ref[1]
    cnt = c_ref[0, pl.ds(j * blk, blk)] + c_ref[1, pl.ds(j * blk, blk)]
    mean = s * (1.0 / jnp.maximum(cnt, 1.0))[:, None]
    acc = jnp.dot(mean, wl_ref[...], preferred_element_type=jnp.float32)
    acc = acc + jnp.dot(h_ref[...], wr_ref[...],
                        preferred_element_type=jnp.float32)
    acc = acc + b_ref[...]
    if relu:
      acc = jnp.maximum(acc, 0.0)
    o_ref[...] = acc

  return pl.pallas_call(
      body,
      grid=(n_pad // blk,),
      in_specs=[
          pl.BlockSpec((2, blk, feat), lambda j: (0, j, 0)),
          pl.BlockSpec((2, n_pad), lambda j: (0, 0)),
          pl.BlockSpec((blk, feat), lambda j: (j, 0)),
          pl.BlockSpec((feat, hidden), lambda j: (0, 0)),
          pl.BlockSpec((feat, hidden), lambda j: (0, 0)),
          pl.BlockSpec((1, hidden), lambda j: (0, 0)),
      ],
      out_specs=pl.BlockSpec((blk, hidden), lambda j: (j, 0)),
      out_shape=jax.ShapeDtypeStruct((n_pad, hidden), jnp.float32),
  )


def kernel(x, edge_index, W1l, W1r, b1, W2l, W2r, b2):
  n, d = x.shape
  h = W1l.shape[0]
  e = edge_index.shape[1]
  gch = NW * CHUNK
  ch = -(-e // gch)
  ch += (-ch) % 16
  e_pad = NW * ch * CHUNK
  pad = e_pad - e
  n_pad = n + PADR
  assert n % NS == 0 and n_pad % NS == 0 and d % 16 == 0

  src = edge_index[0]
  dst = edge_index[1]
  if pad:
    ar = jnp.arange(pad, dtype=jnp.int32)
    src = jnp.concatenate([src, ar % n])
    dst = jnp.concatenate([dst, n + ar % PADR])
  src_r = src.reshape(NW, ch, CHUNK)
  dst_r = dst.reshape(NW, ch, CHUNK)
  x_p = jnp.pad(x, ((0, PADR), (0, 0)))

  s1, cnt = _sc_segsum(n_pad, ch, d, with_cnt=True)(x, src_r, dst_r)
  cnt2 = cnt.reshape(NC, n_pad)
  h1 = _tc_layer(n_pad, d, h, relu=True)(
      s1, cnt2, x_p, W1l.T, W1r.T, b1[None, :])
  (s2,) = _sc_segsum(n_pad, ch, h, with_cnt=False)(h1, src_r, dst_r)
  out = _tc_layer(n_pad, h, h, relu=False)(
      s2, cnt2, h1, W2l.T, W2r.T, b2[None, :])
  return out[:n]


# speedup vs baseline: 1.3211x; 1.3211x over previous
import jax
import jax.numpy as jnp
from jax import lax
from jax.experimental import pallas as pl
from jax.experimental.pallas import tpu as pltpu
from jax.experimental.pallas import tpu_sc as plsc

NC = 2
NS = 16
NW = NC * NS
CHUNK = 128
NSLOT = 2
PADR = 240


def _sc_segsum(n_pad, ch, feat, with_cnt):
  zrows = n_pad // NS
  ib = 16
  assert ch % ib == 0 and ib % NSLOT == 0

  mesh = plsc.VectorSubcoreMesh(
      core_axis_name="c", subcore_axis_name="s",
      num_cores=NC, num_subcores=NS)

  out_type = [jax.ShapeDtypeStruct((NC, n_pad, feat), jnp.float32)]
  scratch = [
      pltpu.VMEM((2, ib, CHUNK), jnp.int32),
      pltpu.VMEM((2, ib, CHUNK), jnp.int32),
      pltpu.VMEM((NSLOT, CHUNK, feat), jnp.float32),
      pltpu.VMEM_SHARED((n_pad, feat), jnp.float32),
      [pltpu.SemaphoreType.DMA] * NSLOT,
      [pltpu.SemaphoreType.DMA] * 2,
  ]
  if with_cnt:
    out_type.append(jax.ShapeDtypeStruct((NC * n_pad,), jnp.float32))
    scratch += [
        pltpu.VMEM((CHUNK,), jnp.float32),
        pltpu.VMEM((1024,), jnp.float32),
        pltpu.VMEM_SHARED((n_pad,), jnp.float32),
    ]

  def body(h_hbm, src_hbm, dst_hbm, s_out, *rest):
    if with_cnt:
      (cnt_out, src_v, dst_v, buf, acc, sem_g, sem_i,
       ones_v, zeros_v, cnt_acc) = rest
    else:
      src_v, dst_v, buf, acc, sem_g, sem_i = rest
      cnt_out = ones_v = zeros_v = cnt_acc = None
    cid = lax.axis_index("c")
    sid = lax.axis_index("s")
    wid = sid * NC + cid
    ngroups = ch // ib

    def gather(bank, l, u):
      return pltpu.make_async_copy(h_hbm.at[src_v.at[bank, l]], buf.at[u],
                                   sem_g[u])

    def idx_load(g, bank):
      return (pltpu.make_async_copy(src_hbm.at[wid, pl.ds(g * ib, ib)],
                                    src_v.at[bank], sem_i[bank]),
              pltpu.make_async_copy(dst_hbm.at[wid, pl.ds(g * ib, ib)],
                                    dst_v.at[bank], sem_i[bank]))

    zero16 = jnp.zeros((16,), jnp.float32)

    def zrow(r, carry):
      def zcol(k, c2):
        buf[0, r, pl.ds(k * 16, 16)] = zero16
        return c2
      return lax.fori_loop(0, feat // 16, zcol, carry)
    lax.fori_loop(0, CHUNK, zrow, 0)

    base = sid * zrows
    nz = zrows // CHUNK

    def zdma(i, carry):
      pltpu.sync_copy(buf.at[0],
                      acc.at[pl.ds(base + i * CHUNK, CHUNK)])
      return carry
    lax.fori_loop(0, nz, zdma, 0)
    assert nz * CHUNK == zrows

    if with_cnt:
      one16 = jnp.ones((16,), jnp.float32)
      for k in range(CHUNK // 16):
        ones_v[pl.ds(k * 16, 16)] = one16

      def zh(i, carry):
        zeros_v[pl.ds(i * 16, 16)] = zero16
        return carry
      lax.fori_loop(0, 1024 // 16, zh, 0)

      @pl.when(sid == 0)
      def _():
        def zc(i, carry):
          pltpu.sync_copy(zeros_v, cnt_acc.at[pl.ds(i * 1024, 1024)])
          return carry
        lax.fori_loop(0, n_pad // 1024, zc, 0)
        assert n_pad % 1024 == 0

    for desc in idx_load(0, 0) + (idx_load(1, 1) if ch > ib else ()):
      desc.start()

    plsc.subcore_barrier()

    def count_chunk(bank, l):
      if with_cnt:
        pltpu.sync_copy(ones_v, cnt_acc.at[dst_v.at[bank, l]], add=True)

    for desc in idx_load(0, 0):
      desc.wait()
    gather(0, 0, 0).start()
    gather(0, 1, 1).start()

    for g in range(ngroups):
      bank = g % 2
      nbank = (g + 1) % 2

      def pair(p, c2, bank=bank, nbank=nbank, g=g):
        for u in range(NSLOT):
          l = NSLOT * p + u
          gather(bank, l, u).wait()
          pltpu.sync_copy(buf.at[u], acc.at[dst_v.at[bank, l]], add=True)
          count_chunk(bank, l)
          lp = l + NSLOT

          @pl.when(lp < ib)
          def _():
            gather(bank, lp, u).start()

          if g + 1 < ngroups:
            @pl.when(lp >= ib)
            def _():
              if u == 0:
                for desc in idx_load(g + 1, nbank):
                  desc.wait()
              gather(nbank, u, u).start()
        return c2
      lax.fori_loop(0, ib // NSLOT, pair, 0)

      if g + 2 < ngroups:
        for desc in idx_load(g + 2, bank):
          desc.start()
    plsc.subcore_barrier()
    pltpu.sync_copy(acc.at[pl.ds(base, zrows)],
                    s_out.at[cid, pl.ds(base, zrows)])
    if with_cnt:
      @pl.when(sid == 0)
      def _():
        pltpu.sync_copy(cnt_acc, cnt_out.at[pl.ds(cid * n_pad, n_pad)])

  return pl.kernel(body, out_type=tuple(out_type), mesh=mesh,
                   scratch_types=tuple(scratch))


def _tc_layer(n_pad, feat, hidden, relu, blk=1024):
  def body(s_ref, c_ref, h_ref, wl_ref, wr_ref, b_ref, o_ref):
    j = pl.program_id(0)
    s = s_ref[0] + s_ref[1]
    cnt = c_ref[0, pl.ds(j * blk, blk)] + c_ref[1, pl.ds(j * blk, blk)]
    mean = s * (1.0 / jnp.maximum(cnt, 1.0))[:, None]
    acc = jnp.dot(mean, wl_ref[...], preferred_element_type=jnp.float32)
    acc = acc + jnp.dot(h_ref[...], wr_ref[...],
                        preferred_element_type=jnp.float32)
    acc = acc + b_ref[...]
    if relu:
      acc = jnp.maximum(acc, 0.0)
    o_ref[...] = acc

  return pl.pallas_call(
      body,
      grid=(n_pad // blk,),
      in_specs=[
          pl.BlockSpec((2, blk, feat), lambda j: (0, j, 0)),
          pl.BlockSpec((2, n_pad), lambda j: (0, 0)),
          pl.BlockSpec((blk, feat), lambda j: (j, 0)),
          pl.BlockSpec((feat, hidden), lambda j: (0, 0)),
          pl.BlockSpec((feat, hidden), lambda j: (0, 0)),
          pl.BlockSpec((1, hidden), lambda j: (0, 0)),
      ],
      out_specs=pl.BlockSpec((blk, hidden), lambda j: (j, 0)),
      out_shape=jax.ShapeDtypeStruct((n_pad, hidden), jnp.float32),
  )


def kernel(x, edge_index, W1l, W1r, b1, W2l, W2r, b2):
  n, d = x.shape
  h = W1l.shape[0]
  e = edge_index.shape[1]
  gch = NW * CHUNK
  ch = -(-e // gch)
  ch += (-ch) % 16
  e_pad = NW * ch * CHUNK
  pad = e_pad - e
  n_pad = n + PADR
  assert n % NS == 0 and n_pad % NS == 0 and d % 16 == 0

  src = edge_index[0]
  dst = edge_index[1]
  if pad:
    ar = jnp.arange(pad, dtype=jnp.int32)
    src = jnp.concatenate([src, ar % n])
    dst = jnp.concatenate([dst, n + ar % PADR])
  src_r = src.reshape(NW, ch, CHUNK)
  dst_r = dst.reshape(NW, ch, CHUNK)
  x_p = jnp.pad(x, ((0, PADR), (0, 0)))

  s1, cnt = _sc_segsum(n_pad, ch, d, with_cnt=True)(x, src_r, dst_r)
  cnt2 = cnt.reshape(NC, n_pad)
  h1 = _tc_layer(n_pad, d, h, relu=True)(
      s1, cnt2, x_p, W1l.T, W1r.T, b1[None, :])
  (s2,) = _sc_segsum(n_pad, ch, h, with_cnt=False)(h1, src_r, dst_r)
  out = _tc_layer(n_pad, h, h, relu=False)(
      s2, cnt2, h1, W2l.T, W2r.T, b2[None, :])
  return out[:n]


# speedup vs baseline: 1.3738x; 1.0399x over previous
import jax
import jax.numpy as jnp
from jax import lax
from jax.experimental import pallas as pl
from jax.experimental.pallas import tpu as pltpu
from jax.experimental.pallas import tpu_sc as plsc

NC = 2
NS = 16
NW = NC * NS
CHUNK = 128
NSLOT = 2
PADR = 240


def _sc_segsum(n_pad, ch, feat, with_cnt):
  zrows = n_pad // NS
  ib = 16
  assert ch % ib == 0 and ib % NSLOT == 0

  mesh = plsc.VectorSubcoreMesh(
      core_axis_name="c", subcore_axis_name="s",
      num_cores=NC, num_subcores=NS)

  out_type = [jax.ShapeDtypeStruct((NC, n_pad, feat), jnp.float32)]
  scratch = [
      pltpu.VMEM((2, ib, CHUNK), jnp.int32),
      pltpu.VMEM((2, ib, CHUNK), jnp.int32),
      pltpu.VMEM((NSLOT, CHUNK, feat), jnp.float32),
      pltpu.VMEM_SHARED((n_pad, feat), jnp.float32),
      [pltpu.SemaphoreType.DMA] * NSLOT,
      [pltpu.SemaphoreType.DMA] * 2,
  ]
  if with_cnt:
    out_type.append(jax.ShapeDtypeStruct((NC * n_pad,), jnp.float32))
    scratch += [
        pltpu.VMEM((CHUNK,), jnp.float32),
        pltpu.VMEM((1024,), jnp.float32),
        pltpu.VMEM_SHARED((n_pad,), jnp.float32),
    ]

  def body(h_hbm, src_hbm, dst_hbm, s_out, *rest):
    if with_cnt:
      (cnt_out, src_v, dst_v, buf, acc, sem_g, sem_i,
       ones_v, zeros_v, cnt_acc) = rest
    else:
      src_v, dst_v, buf, acc, sem_g, sem_i = rest
      cnt_out = ones_v = zeros_v = cnt_acc = None
    cid = lax.axis_index("c")
    sid = lax.axis_index("s")
    wid = sid * NC + cid
    ngroups = ch // ib

    def gather(bank, l, u):
      return pltpu.make_async_copy(h_hbm.at[src_v.at[bank, l]], buf.at[u],
                                   sem_g[u])

    def idx_load(g, bank):
      return (pltpu.make_async_copy(src_hbm.at[wid, pl.ds(g * ib, ib)],
                                    src_v.at[bank], sem_i[bank]),
              pltpu.make_async_copy(dst_hbm.at[wid, pl.ds(g * ib, ib)],
                                    dst_v.at[bank], sem_i[bank]))

    zero16 = jnp.zeros((16,), jnp.float32)

    def zrow(r, carry):
      def zcol(k, c2):
        buf[0, r, pl.ds(k * 16, 16)] = zero16
        return c2
      return lax.fori_loop(0, feat // 16, zcol, carry)
    lax.fori_loop(0, CHUNK, zrow, 0)

    base = sid * zrows
    nz = zrows // CHUNK

    def zdma(i, carry):
      pltpu.sync_copy(buf.at[0],
                      acc.at[pl.ds(base + i * CHUNK, CHUNK)])
      return carry
    lax.fori_loop(0, nz, zdma, 0)
    assert nz * CHUNK == zrows

    if with_cnt:
      one16 = jnp.ones((16,), jnp.float32)
      for k in range(CHUNK // 16):
        ones_v[pl.ds(k * 16, 16)] = one16

      def zh(i, carry):
        zeros_v[pl.ds(i * 16, 16)] = zero16
        return carry
      lax.fori_loop(0, 1024 // 16, zh, 0)

      @pl.when(sid == 0)
      def _():
        def zc(i, carry):
          pltpu.sync_copy(zeros_v, cnt_acc.at[pl.ds(i * 1024, 1024)])
          return carry
        lax.fori_loop(0, n_pad // 1024, zc, 0)
        assert n_pad % 1024 == 0

    for desc in idx_load(0, 0) + (idx_load(1, 1) if ch > ib else ()):
      desc.start()

    plsc.subcore_barrier()

    def count_chunk(bank, l):
      if with_cnt:
        pltpu.sync_copy(ones_v, cnt_acc.at[dst_v.at[bank, l]], add=True)

    for desc in idx_load(0, 0):
      desc.wait()
    gather(0, 0, 0).start()
    gather(0, 1, 1).start()

    for g in range(ngroups):
      bank = g % 2
      nbank = (g + 1) % 2

      def pair(p, c2, bank=bank, nbank=nbank, g=g):
        for u in range(NSLOT):
          l = NSLOT * p + u
          gather(bank, l, u).wait()
          pltpu.sync_copy(buf.at[u], acc.at[dst_v.at[bank, l]], add=True)
          count_chunk(bank, l)
          lp = l + NSLOT

          @pl.when(lp < ib)
          def _():
            gather(bank, lp, u).start()

          if g + 1 < ngroups:
            @pl.when(lp >= ib)
            def _():
              if u == 0:
                for desc in idx_load(g + 1, nbank):
                  desc.wait()
              gather(nbank, u, u).start()
        return c2
      lax.fori_loop(0, ib // NSLOT, pair, 0)

      if g + 2 < ngroups:
        for desc in idx_load(g + 2, bank):
          desc.start()
    plsc.subcore_barrier()
    pltpu.sync_copy(acc.at[pl.ds(base, zrows)],
                    s_out.at[cid, pl.ds(base, zrows)])
    if with_cnt:
      @pl.when(sid == 0)
      def _():
        pltpu.sync_copy(cnt_acc, cnt_out.at[pl.ds(cid * n_pad, n_pad)])

  return pl.kernel(body, out_type=tuple(out_type), mesh=mesh,
                   scratch_types=tuple(scratch))


def _tc_layer(n_rows, n_pad, feat, hidden, relu, blk=2048):

  def body(s_ref, c_ref, h_ref, wl_ref, wr_ref, b_ref, o_ref):
    j = pl.program_id(0)
    s = s_ref[0] + s_ref[1]
    cnt = c_ref[0, pl.ds(j * blk, blk)] + c_ref[1, pl.ds(j * blk, blk)]
    mean = s * (1.0 / jnp.maximum(cnt, 1.0))[:, None]
    acc = jnp.dot(mean, wl_ref[...], preferred_element_type=jnp.float32)
    acc = acc + jnp.dot(h_ref[...], wr_ref[...],
                        preferred_element_type=jnp.float32)
    acc = acc + b_ref[...]
    if relu:
      acc = jnp.maximum(acc, 0.0)
    o_ref[...] = acc

  return pl.pallas_call(
      body,
      grid=(n_pad // blk,),
      in_specs=[
          pl.BlockSpec((2, blk, feat), lambda j: (0, j, 0)),
          pl.BlockSpec((2, n_pad), lambda j: (0, 0)),
          pl.BlockSpec((blk, feat), lambda j: (j, 0)),
          pl.BlockSpec((feat, hidden), lambda j: (0, 0)),
          pl.BlockSpec((feat, hidden), lambda j: (0, 0)),
          pl.BlockSpec((1, hidden), lambda j: (0, 0)),
      ],
      out_specs=pl.BlockSpec((blk, hidden), lambda j: (j, 0)),
      out_shape=jax.ShapeDtypeStruct((n_rows, hidden), jnp.float32),
  )


def kernel(x, edge_index, W1l, W1r, b1, W2l, W2r, b2):
  n, d = x.shape
  h = W1l.shape[0]
  e = edge_index.shape[1]
  gch = NW * CHUNK
  ch = -(-e // gch)
  ch += (-ch) % 16
  e_pad = NW * ch * CHUNK
  pad = e_pad - e
  n_pad = n + PADR
  assert n % NS == 0 and n_pad % NS == 0 and d % 16 == 0

  src = edge_index[0]
  dst = edge_index[1]
  if pad:
    ar = jnp.arange(pad, dtype=jnp.int32)
    src = jnp.concatenate([src, ar % n])
    dst = jnp.concatenate([dst, n + ar % PADR])
  src_r = src.reshape(NW, ch, CHUNK)
  dst_r = dst.reshape(NW, ch, CHUNK)

  s1, cnt = _sc_segsum(n_pad, ch, d, with_cnt=True)(x, src_r, dst_r)
  cnt2 = cnt.reshape(NC, n_pad)
  h1 = _tc_layer(n, n_pad, d, h, relu=True)(
      s1, cnt2, x, W1l.T, W1r.T, b1[None, :])
  (s2,) = _sc_segsum(n_pad, ch, h, with_cnt=False)(h1, src_r, dst_r)
  return _tc_layer(n, n_pad, h, h, relu=False)(
      s2, cnt2, h1, W2l.T, W2r.T, b2[None, :])
